# baseline (speedup 1.0000x reference)
import jax
import jax.numpy as jnp
from jax import lax
from jax.experimental import pallas as pl
from jax.experimental.pallas import tpu as pltpu
from jax.experimental.pallas import tpu_sc as plsc

N_NODES = 10000
D = 128
N_LAYERS = 3
NC, NS = 2, 16
NW = NC * NS
N_PAD = 10240
EK = 128
EPT = 10240
NCHUNK = EPT // EK
E_PAD = NW * EPT
RPT = N_PAD // NS
BLK = 1024
GRID = N_PAD // BLK


def _sc_deg_body(dst_hbm, degp_hbm, idst, ones_v, zeros_v, acc):
    c = lax.axis_index("c")
    s = lax.axis_index("s")
    wid = c * NS + s
    zeros16 = jnp.zeros((16,), jnp.float32)
    ones16 = jnp.ones((16,), jnp.float32)

    def fill_body(i, carry):
        zeros_v[pl.ds(i * 16, 16)] = zeros16
        return carry

    lax.fori_loop(0, RPT // 16, fill_body, 0)

    def ones_body(i, carry):
        ones_v[pl.ds(i * 16, 16)] = ones16
        return carry

    lax.fori_loop(0, EK // 16, ones_body, 0)
    pltpu.sync_copy(zeros_v, acc.at[pl.ds(s * RPT, RPT)])
    plsc.subcore_barrier()

    def chunk_body(j, carry):
        pltpu.sync_copy(dst_hbm.at[wid, j], idst)
        pltpu.sync_copy(ones_v, acc.at[idst], add=True)
        return carry

    lax.fori_loop(0, NCHUNK, chunk_body, 0)
    plsc.subcore_barrier()
    pltpu.sync_copy(acc.at[pl.ds(s * RPT, RPT)],
                    degp_hbm.at[c, pl.ds(s * RPT, RPT)])


NHALF = 2
CPH = NCHUNK // NHALF


def _sc_scatter_body(u_hbm, src_hbm, dst_hbm, s_hbm,
                     isrc, idst, rows0, rows1, acc, gs0, gs1, ss0, ss1):
    c = lax.axis_index("c")
    s = lax.axis_index("s")
    wid = c * NS + s
    zeros16 = jnp.zeros((16,), jnp.float32)

    def zero_body(i, carry):
        for l in range(D // 16):
            rows0[i, pl.ds(l * 16, 16)] = zeros16
        return carry

    lax.fori_loop(0, EK, zero_body, 0)
    for z in range(RPT // EK):
        pltpu.sync_copy(rows0, acc.at[pl.ds(s * RPT + z * EK, EK)])
    plsc.subcore_barrier()

    for h in range(NHALF):
        pltpu.sync_copy(src_hbm.at[wid, pl.ds(h * CPH, CPH)], isrc)
        pltpu.sync_copy(dst_hbm.at[wid, pl.ds(h * CPH, CPH)], idst)
        pltpu.async_copy(u_hbm.at[isrc.at[0]], rows0, gs0)
        pltpu.async_copy(u_hbm.at[isrc.at[1]], rows1, gs1)

        def pair_body(jj, carry):
            j0 = 2 * jj
            j1 = j0 + 1
            pltpu.make_async_copy(u_hbm.at[isrc.at[j0]], rows0, gs0).wait()
            pltpu.async_copy(rows0, acc.at[idst.at[j0]], ss0, add=True)
            pltpu.make_async_copy(u_hbm.at[isrc.at[j1]], rows1, gs1).wait()
            pltpu.async_copy(rows1, acc.at[idst.at[j1]], ss1, add=True)

            @pl.when(jj + 1 < CPH // 2)
            def _():
                pltpu.make_async_copy(rows0, acc.at[idst.at[j0]], ss0).wait()
                pltpu.async_copy(u_hbm.at[isrc.at[j0 + 2]], rows0, gs0)
                pltpu.make_async_copy(rows1, acc.at[idst.at[j1]], ss1).wait()
                pltpu.async_copy(u_hbm.at[isrc.at[j1 + 2]], rows1, gs1)
            return carry

        lax.fori_loop(0, CPH // 2, pair_body, 0)
        pltpu.make_async_copy(rows0, acc.at[idst.at[CPH - 2]], ss0).wait()
        pltpu.make_async_copy(rows1, acc.at[idst.at[CPH - 1]], ss1).wait()
    plsc.subcore_barrier()
    pltpu.sync_copy(acc.at[pl.ds(s * RPT, RPT)],
                    s_hbm.at[c, pl.ds(s * RPT, RPT)])


_sc_calls_cache = []


def _get_sc_calls():
    if _sc_calls_cache:
        return _sc_calls_cache[0]
    mesh = plsc.VectorSubcoreMesh(core_axis_name="c", subcore_axis_name="s",
                                  num_cores=NC, num_subcores=NS)
    deg_call = pl.kernel(
        _sc_deg_body,
        out_type=jax.ShapeDtypeStruct((NC, N_PAD), jnp.float32),
        mesh=mesh,
        scratch_types=[
            pltpu.VMEM((EK,), jnp.int32),
            pltpu.VMEM((EK,), jnp.float32),
            pltpu.VMEM((RPT,), jnp.float32),
            pltpu.VMEM_SHARED((N_PAD,), jnp.float32),
        ],
    )
    scatter_call = pl.kernel(
        _sc_scatter_body,
        out_type=jax.ShapeDtypeStruct((NC, N_PAD, D), jnp.float32),
        mesh=mesh,
        scratch_types=[
            pltpu.VMEM((CPH, EK), jnp.int32),
            pltpu.VMEM((CPH, EK), jnp.int32),
            pltpu.VMEM((EK, D), jnp.float32),
            pltpu.VMEM((EK, D), jnp.float32),
            pltpu.VMEM_SHARED((N_PAD, D), jnp.float32),
            pltpu.SemaphoreType.DMA,
            pltpu.SemaphoreType.DMA,
            pltpu.SemaphoreType.DMA,
            pltpu.SemaphoreType.DMA,
        ],
    )
    _sc_calls_cache.append((deg_call, scatter_call))
    return _sc_calls_cache[0]


def _dinv_from_degp(degp_blk):
    ones = jnp.ones((NC, D), jnp.float32)
    degb = lax.dot_general(degp_blk, ones, (((0,), (0,)), ((), ())),
                           preferred_element_type=jnp.float32)
    return lax.rsqrt(degb + 1.0)


def _tc_embed_body(x_ref, wa_ref, ba_ref, w1_ref, degp_ref, h0_ref, u1_ref):
    h0 = jnp.dot(x_ref[...], wa_ref[...],
                 preferred_element_type=jnp.float32) + ba_ref[...]
    dinv = _dinv_from_degp(degp_ref[...])
    h0_ref[...] = h0
    u1_ref[...] = dinv * jnp.dot(h0, w1_ref[...],
                                 preferred_element_type=jnp.float32)


def _tc_layer_common(s_ref, u_ref, hin_ref, degp_ref,
                     g_ref, be_ref, mu_ref, var_ref, b_ref):
    dinv = _dinv_from_degp(degp_ref[...])
    pre = dinv * (s_ref[0] + s_ref[1] + u_ref[...]) + b_ref[...]
    scale = g_ref[...] * lax.rsqrt(var_ref[...] + 1e-5)
    hbn = (pre - mu_ref[...]) * scale + be_ref[...]
    h = jnp.maximum(hbn, 0.0) + hin_ref[...]
    return h, dinv


def _tc_layer_body(s_ref, u_ref, hin_ref, degp_ref, g_ref, be_ref, mu_ref,
                   var_ref, b_ref, wn_ref, hout_ref, unext_ref):
    h, dinv = _tc_layer_common(s_ref, u_ref, hin_ref, degp_ref,
                               g_ref, be_ref, mu_ref, var_ref, b_ref)
    hout_ref[...] = h
    unext_ref[...] = dinv * jnp.dot(h, wn_ref[...],
                                    preferred_element_type=jnp.float32)


def _tc_final_body(s_ref, u_ref, hin_ref, degp_ref, g_ref, be_ref, mu_ref,
                   var_ref, b_ref, wo_ref, bo_ref, out_ref):
    h, _ = _tc_layer_common(s_ref, u_ref, hin_ref, degp_ref,
                            g_ref, be_ref, mu_ref, var_ref, b_ref)
    out_ref[...] = jnp.dot(h, wo_ref[...],
                           preferred_element_type=jnp.float32) + bo_ref[...]


_row_spec = pl.BlockSpec((BLK, D), lambda i: (i, 0))
_mat_spec = pl.BlockSpec((D, D), lambda i: (0, 0))
_vec_spec = pl.BlockSpec((1, D), lambda i: (0, 0))
_degp_spec = pl.BlockSpec((NC, BLK), lambda i: (0, i))
_s_spec = pl.BlockSpec((NC, BLK, D), lambda i: (0, i, 0))

_embed_call = pl.pallas_call(
    _tc_embed_body,
    grid=(GRID,),
    in_specs=[_row_spec, _mat_spec, _vec_spec, _mat_spec, _degp_spec],
    out_specs=[_row_spec, _row_spec],
    out_shape=[jax.ShapeDtypeStruct((N_PAD, D), jnp.float32)] * 2,
)

_layer_call = pl.pallas_call(
    _tc_layer_body,
    grid=(GRID,),
    in_specs=[_s_spec, _row_spec, _row_spec, _degp_spec,
              _vec_spec, _vec_spec, _vec_spec, _vec_spec, _vec_spec,
              _mat_spec],
    out_specs=[_row_spec, _row_spec],
    out_shape=[jax.ShapeDtypeStruct((N_PAD, D), jnp.float32)] * 2,
)

_final_call = pl.pallas_call(
    _tc_final_body,
    grid=(GRID,),
    in_specs=[_s_spec, _row_spec, _row_spec, _degp_spec,
              _vec_spec, _vec_spec, _vec_spec, _vec_spec, _vec_spec,
              _mat_spec, _vec_spec],
    out_specs=_row_spec,
    out_shape=jax.ShapeDtypeStruct((N_PAD, D), jnp.float32),
)


def kernel(x, edge_index, Wa, ba, conv_W, conv_b, bn_gamma, bn_beta,
           bn_mean, bn_var, Wo, bo):
    n = x.shape[0]
    e = edge_index.shape[1]
    fill = jnp.full((E_PAD - e,), n, jnp.int32)
    srcp = jnp.concatenate([edge_index[0], fill]).reshape(NW, NCHUNK, EK)
    dstp = jnp.concatenate([edge_index[1], fill]).reshape(NW, NCHUNK, EK)
    xp = jnp.zeros((N_PAD, D), jnp.float32).at[:n].set(x)

    deg_call, scatter_call = _get_sc_calls()
    degp = deg_call(dstp)
    h, u = _embed_call(xp, Wa, ba.reshape(1, D), conv_W[0], degp)
    for i in range(N_LAYERS):
        S = scatter_call(u, srcp, dstp)
        bn = (bn_gamma[i].reshape(1, D), bn_beta[i].reshape(1, D),
              bn_mean[i].reshape(1, D), bn_var[i].reshape(1, D),
              conv_b[i].reshape(1, D))
        if i < N_LAYERS - 1:
            h, u = _layer_call(S, u, h, degp, *bn, conv_W[i + 1])
        else:
            out = _final_call(S, u, h, degp, *bn, Wo, bo.reshape(1, D))
    return out[:n]


# speedup vs baseline: 1.0324x; 1.0324x over previous
import jax
import jax.numpy as jnp
from jax import lax
from jax.experimental import pallas as pl
from jax.experimental.pallas import tpu as pltpu
from jax.experimental.pallas import tpu_sc as plsc

N_NODES = 10000
D = 128
N_LAYERS = 3
NC, NS = 2, 16
NW = NC * NS
N_PAD = 10240
EK = 128
EPT = 10240
NCHUNK = EPT // EK
E_PAD = NW * EPT
RPT = N_PAD // NS
BLK = 1024
GRID = N_PAD // BLK


def _sc_deg_body(dst_hbm, degp_hbm, idst, ones_v, zeros_v, acc):
    c = lax.axis_index("c")
    s = lax.axis_index("s")
    wid = c * NS + s
    zeros16 = jnp.zeros((16,), jnp.float32)
    ones16 = jnp.ones((16,), jnp.float32)

    def fill_body(i, carry):
        zeros_v[pl.ds(i * 16, 16)] = zeros16
        return carry

    lax.fori_loop(0, RPT // 16, fill_body, 0)

    def ones_body(i, carry):
        ones_v[pl.ds(i * 16, 16)] = ones16
        return carry

    lax.fori_loop(0, EK // 16, ones_body, 0)
    pltpu.sync_copy(zeros_v, acc.at[pl.ds(s * RPT, RPT)])
    plsc.subcore_barrier()

    def chunk_body(j, carry):
        pltpu.sync_copy(dst_hbm.at[wid, j], idst)
        pltpu.sync_copy(ones_v, acc.at[idst], add=True)
        return carry

    lax.fori_loop(0, NCHUNK, chunk_body, 0)
    plsc.subcore_barrier()
    pltpu.sync_copy(acc.at[pl.ds(s * RPT, RPT)],
                    degp_hbm.at[c, pl.ds(s * RPT, RPT)])


NHALF = 2
CPH = NCHUNK // NHALF


def _sc_scatter_body(u_hbm, src_hbm, dst_hbm, s_hbm,
                     isrc, idst, rows0, rows1, acc, gs0, gs1, ss0, ss1):
    c = lax.axis_index("c")
    s = lax.axis_index("s")
    wid = c * NS + s
    zeros16 = jnp.zeros((16,), jnp.float32)

    def zero_body(i, carry):
        for l in range(D // 16):
            rows0[i, pl.ds(l * 16, 16)] = zeros16
        return carry

    lax.fori_loop(0, EK, zero_body, 0)
    for z in range(RPT // EK):
        pltpu.sync_copy(rows0, acc.at[pl.ds(s * RPT + z * EK, EK)])
    plsc.subcore_barrier()

    for h in range(NHALF):
        pltpu.sync_copy(src_hbm.at[wid, pl.ds(h * CPH, CPH)], isrc)
        pltpu.sync_copy(dst_hbm.at[wid, pl.ds(h * CPH, CPH)], idst)
        def gather2(j, rows, sa, sb):
            pltpu.async_copy(u_hbm.at[isrc.at[j, pl.ds(0, EK // 2)]],
                             rows.at[pl.ds(0, EK // 2)], sa)
            pltpu.async_copy(u_hbm.at[isrc.at[j, pl.ds(EK // 2, EK // 2)]],
                             rows.at[pl.ds(EK // 2, EK // 2)], sb)

        def wait2(j, rows, sa, sb):
            pltpu.make_async_copy(u_hbm.at[isrc.at[j, pl.ds(0, EK // 2)]],
                                  rows.at[pl.ds(0, EK // 2)], sa).wait()
            pltpu.make_async_copy(u_hbm.at[isrc.at[j, pl.ds(EK // 2, EK // 2)]],
                                  rows.at[pl.ds(EK // 2, EK // 2)], sb).wait()

        gather2(0, rows0, gs0, ss0)
        gather2(1, rows1, gs1, ss1)

        def pair_body(jj, carry):
            j0 = 2 * jj
            j1 = j0 + 1
            wait2(j0, rows0, gs0, ss0)
            pltpu.sync_copy(rows0, acc.at[idst.at[j0]], add=True)

            @pl.when(jj + 1 < CPH // 2)
            def _():
                gather2(j0 + 2, rows0, gs0, ss0)

            wait2(j1, rows1, gs1, ss1)
            pltpu.sync_copy(rows1, acc.at[idst.at[j1]], add=True)

            @pl.when(jj + 1 < CPH // 2)
            def _():
                gather2(j1 + 2, rows1, gs1, ss1)
            return carry

        lax.fori_loop(0, CPH // 2, pair_body, 0)
    plsc.subcore_barrier()
    pltpu.sync_copy(acc.at[pl.ds(s * RPT, RPT)],
                    s_hbm.at[c, pl.ds(s * RPT, RPT)])


_sc_calls_cache = []


def _get_sc_calls():
    if _sc_calls_cache:
        return _sc_calls_cache[0]
    mesh = plsc.VectorSubcoreMesh(core_axis_name="c", subcore_axis_name="s",
                                  num_cores=NC, num_subcores=NS)
    deg_call = pl.kernel(
        _sc_deg_body,
        out_type=jax.ShapeDtypeStruct((NC, N_PAD), jnp.float32),
        mesh=mesh,
        scratch_types=[
            pltpu.VMEM((EK,), jnp.int32),
            pltpu.VMEM((EK,), jnp.float32),
            pltpu.VMEM((RPT,), jnp.float32),
            pltpu.VMEM_SHARED((N_PAD,), jnp.float32),
        ],
    )
    scatter_call = pl.kernel(
        _sc_scatter_body,
        out_type=jax.ShapeDtypeStruct((NC, N_PAD, D), jnp.float32),
        mesh=mesh,
        scratch_types=[
            pltpu.VMEM((CPH, EK), jnp.int32),
            pltpu.VMEM((CPH, EK), jnp.int32),
            pltpu.VMEM((EK, D), jnp.float32),
            pltpu.VMEM((EK, D), jnp.float32),
            pltpu.VMEM_SHARED((N_PAD, D), jnp.float32),
            pltpu.SemaphoreType.DMA,
            pltpu.SemaphoreType.DMA,
            pltpu.SemaphoreType.DMA,
            pltpu.SemaphoreType.DMA,
        ],
    )
    _sc_calls_cache.append((deg_call, scatter_call))
    return _sc_calls_cache[0]


def _dinv_from_degp(degp_blk):
    ones = jnp.ones((NC, D), jnp.float32)
    degb = lax.dot_general(degp_blk, ones, (((0,), (0,)), ((), ())),
                           preferred_element_type=jnp.float32)
    return lax.rsqrt(degb + 1.0)


def _tc_embed_body(x_ref, wa_ref, ba_ref, w1_ref, degp_ref, h0_ref, u1_ref):
    h0 = jnp.dot(x_ref[...], wa_ref[...],
                 preferred_element_type=jnp.float32) + ba_ref[...]
    dinv = _dinv_from_degp(degp_ref[...])
    h0_ref[...] = h0
    u1_ref[...] = dinv * jnp.dot(h0, w1_ref[...],
                                 preferred_element_type=jnp.float32)


def _tc_layer_common(s_ref, u_ref, hin_ref, degp_ref,
                     g_ref, be_ref, mu_ref, var_ref, b_ref):
    dinv = _dinv_from_degp(degp_ref[...])
    pre = dinv * (s_ref[0] + s_ref[1] + u_ref[...]) + b_ref[...]
    scale = g_ref[...] * lax.rsqrt(var_ref[...] + 1e-5)
    hbn = (pre - mu_ref[...]) * scale + be_ref[...]
    h = jnp.maximum(hbn, 0.0) + hin_ref[...]
    return h, dinv


def _tc_layer_body(s_ref, u_ref, hin_ref, degp_ref, g_ref, be_ref, mu_ref,
                   var_ref, b_ref, wn_ref, hout_ref, unext_ref):
    h, dinv = _tc_layer_common(s_ref, u_ref, hin_ref, degp_ref,
                               g_ref, be_ref, mu_ref, var_ref, b_ref)
    hout_ref[...] = h
    unext_ref[...] = dinv * jnp.dot(h, wn_ref[...],
                                    preferred_element_type=jnp.float32)


def _tc_final_body(s_ref, u_ref, hin_ref, degp_ref, g_ref, be_ref, mu_ref,
                   var_ref, b_ref, wo_ref, bo_ref, out_ref):
    h, _ = _tc_layer_common(s_ref, u_ref, hin_ref, degp_ref,
                            g_ref, be_ref, mu_ref, var_ref, b_ref)
    out_ref[...] = jnp.dot(h, wo_ref[...],
                           preferred_element_type=jnp.float32) + bo_ref[...]


_row_spec = pl.BlockSpec((BLK, D), lambda i: (i, 0))
_mat_spec = pl.BlockSpec((D, D), lambda i: (0, 0))
_vec_spec = pl.BlockSpec((1, D), lambda i: (0, 0))
_degp_spec = pl.BlockSpec((NC, BLK), lambda i: (0, i))
_s_spec = pl.BlockSpec((NC, BLK, D), lambda i: (0, i, 0))

_embed_call = pl.pallas_call(
    _tc_embed_body,
    grid=(GRID,),
    in_specs=[_row_spec, _mat_spec, _vec_spec, _mat_spec, _degp_spec],
    out_specs=[_row_spec, _row_spec],
    out_shape=[jax.ShapeDtypeStruct((N_PAD, D), jnp.float32)] * 2,
)

_layer_call = pl.pallas_call(
    _tc_layer_body,
    grid=(GRID,),
    in_specs=[_s_spec, _row_spec, _row_spec, _degp_spec,
              _vec_spec, _vec_spec, _vec_spec, _vec_spec, _vec_spec,
              _mat_spec],
    out_specs=[_row_spec, _row_spec],
    out_shape=[jax.ShapeDtypeStruct((N_PAD, D), jnp.float32)] * 2,
)

_final_call = pl.pallas_call(
    _tc_final_body,
    grid=(GRID,),
    in_specs=[_s_spec, _row_spec, _row_spec, _degp_spec,
              _vec_spec, _vec_spec, _vec_spec, _vec_spec, _vec_spec,
              _mat_spec, _vec_spec],
    out_specs=_row_spec,
    out_shape=jax.ShapeDtypeStruct((N_PAD, D), jnp.float32),
)


def kernel(x, edge_index, Wa, ba, conv_W, conv_b, bn_gamma, bn_beta,
           bn_mean, bn_var, Wo, bo):
    n = x.shape[0]
    e = edge_index.shape[1]
    fill = jnp.full((E_PAD - e,), n, jnp.int32)
    srcp = jnp.concatenate([edge_index[0], fill]).reshape(NW, NCHUNK, EK)
    dstp = jnp.concatenate([edge_index[1], fill]).reshape(NW, NCHUNK, EK)
    xp = jnp.zeros((N_PAD, D), jnp.float32).at[:n].set(x)

    deg_call, scatter_call = _get_sc_calls()
    degp = deg_call(dstp)
    h, u = _embed_call(xp, Wa, ba.reshape(1, D), conv_W[0], degp)
    for i in range(N_LAYERS):
        S = scatter_call(u, srcp, dstp)
        bn = (bn_gamma[i].reshape(1, D), bn_beta[i].reshape(1, D),
              bn_mean[i].reshape(1, D), bn_var[i].reshape(1, D),
              conv_b[i].reshape(1, D))
        if i < N_LAYERS - 1:
            h, u = _layer_call(S, u, h, degp, *bn, conv_W[i + 1])
        else:
            out = _final_call(S, u, h, degp, *bn, Wo, bo.reshape(1, D))
    return out[:n]


# speedup vs baseline: 1.1140x; 1.0790x over previous
import jax
import jax.numpy as jnp
from jax import lax
from jax.experimental import pallas as pl
from jax.experimental.pallas import tpu as pltpu
from jax.experimental.pallas import tpu_sc as plsc

N_NODES = 10000
D = 128
N_LAYERS = 3
NC, NS = 2, 16
NW = NC * NS
N_PAD = 10240
EK = 128
EPT = 10240
NCHUNK = EPT // EK
E_PAD = NW * EPT
RPT = N_PAD // NS
BLK = 1024
GRID = N_PAD // BLK


def _sc_deg_body(dst_hbm, degp_hbm, idst, ones_v, zeros_v, acc):
    c = lax.axis_index("c")
    s = lax.axis_index("s")
    wid = c * NS + s
    zeros16 = jnp.zeros((16,), jnp.float32)
    ones16 = jnp.ones((16,), jnp.float32)

    def fill_body(i, carry):
        zeros_v[pl.ds(i * 16, 16)] = zeros16
        return carry

    lax.fori_loop(0, RPT // 16, fill_body, 0)

    def ones_body(i, carry):
        ones_v[pl.ds(i * 16, 16)] = ones16
        return carry

    lax.fori_loop(0, EK // 16, ones_body, 0)
    pltpu.sync_copy(zeros_v, acc.at[pl.ds(s * RPT, RPT)])
    plsc.subcore_barrier()

    def chunk_body(j, carry):
        pltpu.sync_copy(dst_hbm.at[wid * NCHUNK + j], idst)
        pltpu.sync_copy(ones_v, acc.at[idst], add=True)
        return carry

    lax.fori_loop(0, NCHUNK, chunk_body, 0)
    plsc.subcore_barrier()
    pltpu.sync_copy(acc.at[pl.ds(s * RPT, RPT)],
                    degp_hbm.at[c, pl.ds(s * RPT, RPT)])


TOT_CHUNKS = E_PAD // EK
NCH0 = 128
NCH1 = TOT_CHUNKS // NS - NCH0
CPH = 32


def _sc_scatter_body(u_hbm, src_hbm, dst_hbm, s_hbm,
                     isrc, idst, rows0, rows1, acc, gs0, gs1):
    c = lax.axis_index("c")
    s = lax.axis_index("s")
    zeros16 = jnp.zeros((16,), jnp.float32)

    def zero_body(i, carry):
        for l in range(D // 16):
            rows0[i, pl.ds(l * 16, 16)] = zeros16
        return carry

    lax.fori_loop(0, EK, zero_body, 0)
    for z in range(RPT // EK):
        pltpu.sync_copy(rows0, acc.at[pl.ds(s * RPT + z * EK, EK)])
    plsc.subcore_barrier()

    nch = jnp.where(c == 0, NCH0, NCH1)
    base_chunk = jnp.where(c == 0, s * NCH0, NS * NCH0 + s * NCH1)

    def group_body(g, carry):
        gbase = base_chunk + g * CPH
        pltpu.sync_copy(src_hbm.at[pl.ds(gbase, CPH)], isrc)
        pltpu.sync_copy(dst_hbm.at[pl.ds(gbase, CPH)], idst)
        pltpu.async_copy(u_hbm.at[isrc.at[0]], rows0, gs0)

        def pair_body(jj, carry2):
            j0 = 2 * jj
            pltpu.async_copy(u_hbm.at[isrc.at[j0 + 1]], rows1, gs1)
            pltpu.make_async_copy(u_hbm.at[isrc.at[j0]], rows0, gs0).wait()
            pltpu.sync_copy(rows0, acc.at[idst.at[j0]], add=True)

            @pl.when(jj + 1 < CPH // 2)
            def _():
                pltpu.async_copy(u_hbm.at[isrc.at[j0 + 2]], rows0, gs0)

            pltpu.make_async_copy(u_hbm.at[isrc.at[j0 + 1]], rows1, gs1).wait()
            pltpu.sync_copy(rows1, acc.at[idst.at[j0 + 1]], add=True)
            return carry2

        lax.fori_loop(0, CPH // 2, pair_body, 0)
        return carry

    lax.fori_loop(0, nch // CPH, group_body, 0)
    plsc.subcore_barrier()
    pltpu.sync_copy(acc.at[pl.ds(s * RPT, RPT)],
                    s_hbm.at[c, pl.ds(s * RPT, RPT)])


_sc_calls_cache = []


def _get_sc_calls():
    if _sc_calls_cache:
        return _sc_calls_cache[0]
    mesh = plsc.VectorSubcoreMesh(core_axis_name="c", subcore_axis_name="s",
                                  num_cores=NC, num_subcores=NS)
    deg_call = pl.kernel(
        _sc_deg_body,
        out_type=jax.ShapeDtypeStruct((NC, N_PAD), jnp.float32),
        mesh=mesh,
        scratch_types=[
            pltpu.VMEM((EK,), jnp.int32),
            pltpu.VMEM((EK,), jnp.float32),
            pltpu.VMEM((RPT,), jnp.float32),
            pltpu.VMEM_SHARED((N_PAD,), jnp.float32),
        ],
    )
    scatter_call = pl.kernel(
        _sc_scatter_body,
        out_type=jax.ShapeDtypeStruct((NC, N_PAD, D), jnp.float32),
        mesh=mesh,
        scratch_types=[
            pltpu.VMEM((CPH, EK), jnp.int32),
            pltpu.VMEM((CPH, EK), jnp.int32),
            pltpu.VMEM((EK, D), jnp.float32),
            pltpu.VMEM((EK, D), jnp.float32),
            pltpu.VMEM_SHARED((N_PAD, D), jnp.float32),
            pltpu.SemaphoreType.DMA,
            pltpu.SemaphoreType.DMA,
        ],
    )
    _sc_calls_cache.append((deg_call, scatter_call))
    return _sc_calls_cache[0]


def _dinv_from_degp(degp_blk):
    ones = jnp.ones((NC, D), jnp.float32)
    degb = lax.dot_general(degp_blk, ones, (((0,), (0,)), ((), ())),
                           preferred_element_type=jnp.float32)
    return lax.rsqrt(degb + 1.0)


def _tc_embed_body(x_ref, wa_ref, ba_ref, w1_ref, degp_ref, h0_ref, u1_ref):
    h0 = jnp.dot(x_ref[...], wa_ref[...],
                 preferred_element_type=jnp.float32) + ba_ref[...]
    dinv = _dinv_from_degp(degp_ref[...])
    h0_ref[...] = h0
    u1_ref[...] = dinv * jnp.dot(h0, w1_ref[...],
                                 preferred_element_type=jnp.float32)


def _tc_layer_common(s_ref, u_ref, hin_ref, degp_ref,
                     g_ref, be_ref, mu_ref, var_ref, b_ref):
    dinv = _dinv_from_degp(degp_ref[...])
    pre = dinv * (s_ref[0] + s_ref[1] + u_ref[...]) + b_ref[...]
    scale = g_ref[...] * lax.rsqrt(var_ref[...] + 1e-5)
    hbn = (pre - mu_ref[...]) * scale + be_ref[...]
    h = jnp.maximum(hbn, 0.0) + hin_ref[...]
    return h, dinv


def _tc_layer_body(s_ref, u_ref, hin_ref, degp_ref, g_ref, be_ref, mu_ref,
                   var_ref, b_ref, wn_ref, hout_ref, unext_ref):
    h, dinv = _tc_layer_common(s_ref, u_ref, hin_ref, degp_ref,
                               g_ref, be_ref, mu_ref, var_ref, b_ref)
    hout_ref[...] = h
    unext_ref[...] = dinv * jnp.dot(h, wn_ref[...],
                                    preferred_element_type=jnp.float32)


def _tc_final_body(s_ref, u_ref, hin_ref, degp_ref, g_ref, be_ref, mu_ref,
                   var_ref, b_ref, wo_ref, bo_ref, out_ref):
    h, _ = _tc_layer_common(s_ref, u_ref, hin_ref, degp_ref,
                            g_ref, be_ref, mu_ref, var_ref, b_ref)
    out_ref[...] = jnp.dot(h, wo_ref[...],
                           preferred_element_type=jnp.float32) + bo_ref[...]


_row_spec = pl.BlockSpec((BLK, D), lambda i: (i, 0))
_mat_spec = pl.BlockSpec((D, D), lambda i: (0, 0))
_vec_spec = pl.BlockSpec((1, D), lambda i: (0, 0))
_degp_spec = pl.BlockSpec((NC, BLK), lambda i: (0, i))
_s_spec = pl.BlockSpec((NC, BLK, D), lambda i: (0, i, 0))

_embed_call = pl.pallas_call(
    _tc_embed_body,
    grid=(GRID,),
    in_specs=[_row_spec, _mat_spec, _vec_spec, _mat_spec, _degp_spec],
    out_specs=[_row_spec, _row_spec],
    out_shape=[jax.ShapeDtypeStruct((N_PAD, D), jnp.float32)] * 2,
)

_layer_call = pl.pallas_call(
    _tc_layer_body,
    grid=(GRID,),
    in_specs=[_s_spec, _row_spec, _row_spec, _degp_spec,
              _vec_spec, _vec_spec, _vec_spec, _vec_spec, _vec_spec,
              _mat_spec],
    out_specs=[_row_spec, _row_spec],
    out_shape=[jax.ShapeDtypeStruct((N_PAD, D), jnp.float32)] * 2,
)

_final_call = pl.pallas_call(
    _tc_final_body,
    grid=(GRID,),
    in_specs=[_s_spec, _row_spec, _row_spec, _degp_spec,
              _vec_spec, _vec_spec, _vec_spec, _vec_spec, _vec_spec,
              _mat_spec, _vec_spec],
    out_specs=_row_spec,
    out_shape=jax.ShapeDtypeStruct((N_PAD, D), jnp.float32),
)


def kernel(x, edge_index, Wa, ba, conv_W, conv_b, bn_gamma, bn_beta,
           bn_mean, bn_var, Wo, bo):
    n = x.shape[0]
    e = edge_index.shape[1]
    fill = jnp.full((E_PAD - e,), n, jnp.int32)
    srcp = jnp.concatenate([edge_index[0], fill]).reshape(TOT_CHUNKS, EK)
    dstp = jnp.concatenate([edge_index[1], fill]).reshape(TOT_CHUNKS, EK)
    xp = jnp.zeros((N_PAD, D), jnp.float32).at[:n].set(x)

    deg_call, scatter_call = _get_sc_calls()
    degp = deg_call(dstp)
    h, u = _embed_call(xp, Wa, ba.reshape(1, D), conv_W[0], degp)
    for i in range(N_LAYERS):
        S = scatter_call(u, srcp, dstp)
        bn = (bn_gamma[i].reshape(1, D), bn_beta[i].reshape(1, D),
              bn_mean[i].reshape(1, D), bn_var[i].reshape(1, D),
              conv_b[i].reshape(1, D))
        if i < N_LAYERS - 1:
            h, u = _layer_call(S, u, h, degp, *bn, conv_W[i + 1])
        else:
            out = _final_call(S, u, h, degp, *bn, Wo, bo.reshape(1, D))
    return out[:n]


# speedup vs baseline: 1.2160x; 1.0916x over previous
import jax
import jax.numpy as jnp
from jax import lax
from jax.experimental import pallas as pl
from jax.experimental.pallas import tpu as pltpu
from jax.experimental.pallas import tpu_sc as plsc

N_NODES = 10000
D = 128
N_LAYERS = 3
NC, NS = 2, 16
NW = NC * NS
N_PAD = 10240
EK = 128
EPT = 10240
NCHUNK = EPT // EK
E_PAD = NW * EPT
RPT = N_PAD // NS
BLK = 1024
GRID = N_PAD // BLK


def _sc_deg_body(dst_hbm, degp_hbm, idst, ones_v, zeros_v, acc):
    c = lax.axis_index("c")
    s = lax.axis_index("s")
    wid = c * NS + s
    zeros16 = jnp.zeros((16,), jnp.float32)
    ones16 = jnp.ones((16,), jnp.float32)

    def fill_body(i, carry):
        zeros_v[pl.ds(i * 16, 16)] = zeros16
        return carry

    lax.fori_loop(0, RPT // 16, fill_body, 0)

    def ones_body(i, carry):
        ones_v[pl.ds(i * 16, 16)] = ones16
        return carry

    lax.fori_loop(0, EK // 16, ones_body, 0)
    pltpu.sync_copy(zeros_v, acc.at[pl.ds(s * RPT, RPT)])
    plsc.subcore_barrier()

    def chunk_body(j, carry):
        pltpu.sync_copy(dst_hbm.at[wid * NCHUNK + j], idst)
        pltpu.sync_copy(ones_v, acc.at[idst], add=True)
        return carry

    lax.fori_loop(0, NCHUNK, chunk_body, 0)
    plsc.subcore_barrier()
    pltpu.sync_copy(acc.at[pl.ds(s * RPT, RPT)],
                    degp_hbm.at[c, pl.ds(s * RPT, RPT)])


TOT_CHUNKS = E_PAD // EK
NCH0 = 144
NCH1 = TOT_CHUNKS // NS - NCH0
CPH = 16


def _sc_scatter_body(u_hbm, src_hbm, dst_hbm, s_hbm,
                     isrc, idst, rows0, rows1, acc, gs0, gs1):
    c = lax.axis_index("c")
    s = lax.axis_index("s")
    zeros16 = jnp.zeros((16,), jnp.float32)

    def zero_body(i, carry):
        for l in range(D // 16):
            rows0[i, pl.ds(l * 16, 16)] = zeros16
        return carry

    lax.fori_loop(0, EK, zero_body, 0)
    for z in range(RPT // EK):
        pltpu.sync_copy(rows0, acc.at[pl.ds(s * RPT + z * EK, EK)])
    plsc.subcore_barrier()

    nch = jnp.where(c == 0, NCH0, NCH1)
    base_chunk = jnp.where(c == 0, s * NCH0, NS * NCH0 + s * NCH1)

    def group_body(g, carry):
        gbase = base_chunk + g * CPH
        pltpu.sync_copy(src_hbm.at[pl.ds(gbase, CPH)], isrc)
        pltpu.sync_copy(dst_hbm.at[pl.ds(gbase, CPH)], idst)
        pltpu.async_copy(u_hbm.at[isrc.at[0]], rows0, gs0)

        def pair_body(jj, carry2):
            j0 = 2 * jj
            pltpu.async_copy(u_hbm.at[isrc.at[j0 + 1]], rows1, gs1)
            pltpu.make_async_copy(u_hbm.at[isrc.at[j0]], rows0, gs0).wait()
            pltpu.sync_copy(rows0, acc.at[idst.at[j0]], add=True)

            @pl.when(jj + 1 < CPH // 2)
            def _():
                pltpu.async_copy(u_hbm.at[isrc.at[j0 + 2]], rows0, gs0)

            pltpu.make_async_copy(u_hbm.at[isrc.at[j0 + 1]], rows1, gs1).wait()
            pltpu.sync_copy(rows1, acc.at[idst.at[j0 + 1]], add=True)
            return carry2

        lax.fori_loop(0, CPH // 2, pair_body, 0)
        return carry

    lax.fori_loop(0, nch // CPH, group_body, 0)
    plsc.subcore_barrier()
    pltpu.sync_copy(acc.at[pl.ds(s * RPT, RPT)],
                    s_hbm.at[c, pl.ds(s * RPT, RPT)])


_sc_calls_cache = []


def _get_sc_calls():
    if _sc_calls_cache:
        return _sc_calls_cache[0]
    mesh = plsc.VectorSubcoreMesh(core_axis_name="c", subcore_axis_name="s",
                                  num_cores=NC, num_subcores=NS)
    deg_call = pl.kernel(
        _sc_deg_body,
        out_type=jax.ShapeDtypeStruct((NC, N_PAD), jnp.float32),
        mesh=mesh,
        scratch_types=[
            pltpu.VMEM((EK,), jnp.int32),
            pltpu.VMEM((EK,), jnp.float32),
            pltpu.VMEM((RPT,), jnp.float32),
            pltpu.VMEM_SHARED((N_PAD,), jnp.float32),
        ],
    )
    scatter_call = pl.kernel(
        _sc_scatter_body,
        out_type=jax.ShapeDtypeStruct((NC, N_PAD, D), jnp.float32),
        mesh=mesh,
        scratch_types=[
            pltpu.VMEM((CPH, EK), jnp.int32),
            pltpu.VMEM((CPH, EK), jnp.int32),
            pltpu.VMEM((EK, D), jnp.float32),
            pltpu.VMEM((EK, D), jnp.float32),
            pltpu.VMEM_SHARED((N_PAD, D), jnp.float32),
            pltpu.SemaphoreType.DMA,
            pltpu.SemaphoreType.DMA,
        ],
    )
    _sc_calls_cache.append((deg_call, scatter_call))
    return _sc_calls_cache[0]


def _dinv_from_degp(degp_blk):
    ones = jnp.ones((NC, D), jnp.float32)
    degb = lax.dot_general(degp_blk, ones, (((0,), (0,)), ((), ())),
                           preferred_element_type=jnp.float32)
    return lax.rsqrt(degb + 1.0)


def _tc_embed_body(x_ref, wa_ref, ba_ref, w1_ref, degp_ref, h0_ref, u1_ref):
    h0 = jnp.dot(x_ref[...], wa_ref[...],
                 preferred_element_type=jnp.float32) + ba_ref[...]
    dinv = _dinv_from_degp(degp_ref[...])
    h0_ref[...] = h0
    u1_ref[...] = dinv * jnp.dot(h0, w1_ref[...],
                                 preferred_element_type=jnp.float32)


def _tc_layer_common(s_ref, u_ref, hin_ref, degp_ref,
                     g_ref, be_ref, mu_ref, var_ref, b_ref):
    dinv = _dinv_from_degp(degp_ref[...])
    pre = dinv * (s_ref[0] + s_ref[1] + u_ref[...]) + b_ref[...]
    scale = g_ref[...] * lax.rsqrt(var_ref[...] + 1e-5)
    hbn = (pre - mu_ref[...]) * scale + be_ref[...]
    h = jnp.maximum(hbn, 0.0) + hin_ref[...]
    return h, dinv


def _tc_layer_body(s_ref, u_ref, hin_ref, degp_ref, g_ref, be_ref, mu_ref,
                   var_ref, b_ref, wn_ref, hout_ref, unext_ref):
    h, dinv = _tc_layer_common(s_ref, u_ref, hin_ref, degp_ref,
                               g_ref, be_ref, mu_ref, var_ref, b_ref)
    hout_ref[...] = h
    unext_ref[...] = dinv * jnp.dot(h, wn_ref[...],
                                    preferred_element_type=jnp.float32)


def _tc_final_body(s_ref, u_ref, hin_ref, degp_ref, g_ref, be_ref, mu_ref,
                   var_ref, b_ref, wo_ref, bo_ref, out_ref):
    h, _ = _tc_layer_common(s_ref, u_ref, hin_ref, degp_ref,
                            g_ref, be_ref, mu_ref, var_ref, b_ref)
    out_ref[...] = jnp.dot(h, wo_ref[...],
                           preferred_element_type=jnp.float32) + bo_ref[...]


_row_spec = pl.BlockSpec((BLK, D), lambda i: (i, 0))
_mat_spec = pl.BlockSpec((D, D), lambda i: (0, 0))
_vec_spec = pl.BlockSpec((1, D), lambda i: (0, 0))
_degp_spec = pl.BlockSpec((NC, BLK), lambda i: (0, i))
_s_spec = pl.BlockSpec((NC, BLK, D), lambda i: (0, i, 0))

_embed_call = pl.pallas_call(
    _tc_embed_body,
    grid=(GRID,),
    in_specs=[_row_spec, _mat_spec, _vec_spec, _mat_spec, _degp_spec],
    out_specs=[_row_spec, _row_spec],
    out_shape=[jax.ShapeDtypeStruct((N_PAD, D), jnp.float32)] * 2,
)

_layer_call = pl.pallas_call(
    _tc_layer_body,
    grid=(GRID,),
    in_specs=[_s_spec, _row_spec, _row_spec, _degp_spec,
              _vec_spec, _vec_spec, _vec_spec, _vec_spec, _vec_spec,
              _mat_spec],
    out_specs=[_row_spec, _row_spec],
    out_shape=[jax.ShapeDtypeStruct((N_PAD, D), jnp.float32)] * 2,
)

_final_call = pl.pallas_call(
    _tc_final_body,
    grid=(GRID,),
    in_specs=[_s_spec, _row_spec, _row_spec, _degp_spec,
              _vec_spec, _vec_spec, _vec_spec, _vec_spec, _vec_spec,
              _mat_spec, _vec_spec],
    out_specs=_row_spec,
    out_shape=jax.ShapeDtypeStruct((N_PAD, D), jnp.float32),
)


def kernel(x, edge_index, Wa, ba, conv_W, conv_b, bn_gamma, bn_beta,
           bn_mean, bn_var, Wo, bo):
    n = x.shape[0]
    e = edge_index.shape[1]
    fill = jnp.full((E_PAD - e,), n, jnp.int32)
    srcp = jnp.concatenate([edge_index[0], fill]).reshape(TOT_CHUNKS, EK)
    dstp = jnp.concatenate([edge_index[1], fill]).reshape(TOT_CHUNKS, EK)
    xp = jnp.zeros((N_PAD, D), jnp.float32).at[:n].set(x)

    deg_call, scatter_call = _get_sc_calls()
    degp = deg_call(dstp)
    h, u = _embed_call(xp, Wa, ba.reshape(1, D), conv_W[0], degp)
    for i in range(N_LAYERS):
        S = scatter_call(u, srcp, dstp)
        bn = (bn_gamma[i].reshape(1, D), bn_beta[i].reshape(1, D),
              bn_mean[i].reshape(1, D), bn_var[i].reshape(1, D),
              conv_b[i].reshape(1, D))
        if i < N_LAYERS - 1:
            h, u = _layer_call(S, u, h, degp, *bn, conv_W[i + 1])
        else:
            out = _final_call(S, u, h, degp, *bn, Wo, bo.reshape(1, D))
    return out[:n]
